# baseline (speedup 1.0000x reference)
import functools

import jax
import jax.numpy as jnp
from jax import lax
from jax.experimental import pallas as pl
from jax.experimental.pallas import tpu as pltpu
from jax.experimental.pallas import tpu_sc as plsc

N = 10000
E = 320000
D = 128
C = 64
CH2 = C // 2

NSC = 2
NTILE = 16
EPT = E // NTILE
CH = 128
NCH = EPT // CH + 1
EPP = NCH * CH - EPT
NPAD = 10240
RPT = NPAD // NTILE
DEAD = N + 200

_MESH = plsc.VectorSubcoreMesh(
    core_axis_name="c", subcore_axis_name="s",
    num_cores=NSC, num_subcores=NTILE)


def _rsqrt16(x):
    xi = plsc.bitcast(x, jnp.int32)
    yi = jnp.int32(0x5F3759DF) - lax.shift_right_logical(xi, 1)
    y = plsc.bitcast(yi, jnp.float32)
    for _ in range(3):
        y = y * (1.5 - 0.5 * x * y * y)
    return y


@functools.partial(
    pl.kernel,
    out_type=(jax.ShapeDtypeStruct((N, C), jnp.float32),
              jax.ShapeDtypeStruct((NSC, NPAD, CH2), jnp.float32)),
    mesh=_MESH,
    compiler_params=pltpu.CompilerParams(use_tc_tiling_on_sc=False,
                                         needs_layout_passes=False),
    scratch_types=[
        pltpu.VMEM((NCH, CH), jnp.int32),
        pltpu.VMEM((NCH, CH), jnp.int32),
        pltpu.VMEM((CH,), jnp.float32),
        pltpu.VMEM((2, CH, CH2), jnp.float32),
        pltpu.VMEM((RPT, CH2), jnp.float32),
        pltpu.VMEM((RPT,), jnp.float32),
        pltpu.VMEM((RPT,), jnp.float32),
        pltpu.VMEM((RPT,), jnp.float32),
        pltpu.VMEM((CH2,), jnp.float32),
        pltpu.VMEM_SHARED((NPAD,), jnp.float32),
        pltpu.VMEM_SHARED((NPAD, CH2), jnp.float32),
        pltpu.VMEM_SHARED((NPAD, CH2), jnp.float32),
        pltpu.SemaphoreType.DMA,
        pltpu.SemaphoreType.DMA,
    ],
)
def _sc_all(y0_hbm, row_hbm, col_hbm, ones_hbm, z1_hbm, b_hbm, out_hbm,
            uh_hbm, row_v, col_v, ones_v, pay, buf, degb, dinvb, d2b, bloc,
            deg_sp, u_sp, acc, sem0, sem1):
    c = lax.axis_index("c")
    s = lax.axis_index("s")
    base = s * RPT

    pltpu.sync_copy(row_hbm.at[s], row_v)
    pltpu.sync_copy(col_hbm.at[s], col_v)
    pltpu.sync_copy(ones_hbm, ones_v)
    pltpu.sync_copy(b_hbm.at[c], bloc)
    pltpu.sync_copy(z1_hbm.at[pl.ds(base, RPT)], deg_sp.at[pl.ds(base, RPT)])
    pltpu.sync_copy(y0_hbm.at[c, pl.ds(base, RPT)], buf)
    plsc.subcore_barrier()

    def dchunk(j, carry):
        pltpu.async_copy(ones_v, deg_sp.at[col_v.at[j]], sem0, add=True)
        return carry

    lax.fori_loop(0, NCH, dchunk, 0)

    def ddrain(j, carry):
        pltpu.make_async_copy(ones_v, deg_sp.at[col_v.at[0]], sem0).wait()
        return carry

    lax.fori_loop(0, NCH, ddrain, 0)
    plsc.subcore_barrier()

    pltpu.sync_copy(deg_sp.at[pl.ds(base, RPT)], degb)

    def newton(k, carry):
        x = degb[pl.ds(k * 16, 16)] + 1.0
        y = _rsqrt16(x)
        dinvb[pl.ds(k * 16, 16)] = y
        d2b[pl.ds(k * 16, 16)] = y * y
        return carry

    lax.fori_loop(0, RPT // 16, newton, 0)

    def uscale(g, carry):
        r0 = g * 16
        dv = dinvb[pl.ds(r0, 16)]
        for l in range(16):
            r = r0 + l
            buf[r, pl.ds(0, 16)] = buf[r, pl.ds(0, 16)] * dv[l]
            buf[r, pl.ds(16, 16)] = buf[r, pl.ds(16, 16)] * dv[l]
        return carry

    lax.fori_loop(0, RPT // 16, uscale, 0)
    pltpu.sync_copy(buf, u_sp.at[pl.ds(base, RPT)])
    pltpu.sync_copy(buf, uh_hbm.at[c, pl.ds(base, RPT)])
    pltpu.sync_copy(buf, acc.at[pl.ds(base, RPT)])
    plsc.subcore_barrier()

    uh = uh_hbm.at[c]

    def hop():
        pltpu.async_copy(uh.at[row_v.at[0]], pay.at[0], sem0)

        def pair(i, carry):
            j = i * 2
            pltpu.async_copy(u_sp.at[row_v.at[j + 1]], pay.at[1], sem1)
            pltpu.make_async_copy(uh.at[row_v.at[j]], pay.at[0],
                                  sem0).wait()
            pltpu.sync_copy(pay.at[0], acc.at[col_v.at[j]], add=True)
            pltpu.async_copy(uh.at[row_v.at[j + 2]], pay.at[0], sem0)
            pltpu.make_async_copy(u_sp.at[row_v.at[j + 1]], pay.at[1],
                                  sem1).wait()
            pltpu.sync_copy(pay.at[1], acc.at[col_v.at[j + 1]], add=True)
            return carry

        lax.fori_loop(0, (NCH - 1) // 2, pair, 0)
        pltpu.make_async_copy(uh.at[row_v.at[NCH - 1]], pay.at[0],
                              sem0).wait()
        pltpu.sync_copy(pay.at[0], acc.at[col_v.at[NCH - 1]], add=True)
        plsc.subcore_barrier()

    assert NCH % 2 == 1
    hop()

    pltpu.sync_copy(acc.at[pl.ds(base, RPT)], buf)

    def mscale(g, carry):
        r0 = g * 16
        dv = d2b[pl.ds(r0, 16)]
        for l in range(16):
            r = r0 + l
            buf[r, pl.ds(0, 16)] = buf[r, pl.ds(0, 16)] * dv[l]
            buf[r, pl.ds(16, 16)] = buf[r, pl.ds(16, 16)] * dv[l]
        return carry

    lax.fori_loop(0, RPT // 16, mscale, 0)
    pltpu.sync_copy(buf, u_sp.at[pl.ds(base, RPT)])
    pltpu.sync_copy(buf, uh_hbm.at[c, pl.ds(base, RPT)])
    pltpu.sync_copy(buf, acc.at[pl.ds(base, RPT)])
    plsc.subcore_barrier()

    hop()

    pltpu.sync_copy(acc.at[pl.ds(base, RPT)], buf)
    b0 = bloc[pl.ds(0, 16)]
    b1 = bloc[pl.ds(16, 16)]

    def oscale(g, carry):
        r0 = g * 16
        dv = dinvb[pl.ds(r0, 16)]
        for l in range(16):
            r = r0 + l
            buf[r, pl.ds(0, 16)] = buf[r, pl.ds(0, 16)] * dv[l] + b0
            buf[r, pl.ds(16, 16)] = buf[r, pl.ds(16, 16)] * dv[l] + b1
        return carry

    lax.fori_loop(0, RPT // 16, oscale, 0)
    last = N - (NTILE - 1) * RPT

    @pl.when(s < NTILE - 1)
    def _():
        pltpu.sync_copy(buf,
                        out_hbm.at[pl.ds(base, RPT), pl.ds(c * CH2, CH2)])

    @pl.when(s == NTILE - 1)
    def _():
        pltpu.sync_copy(buf.at[pl.ds(0, last)],
                        out_hbm.at[pl.ds(base, last), pl.ds(c * CH2, CH2)])


BLK = 80
GRID = N // BLK


def _mm_body(x_ref, w_ref, y_ref):
    y = lax.dot_general(x_ref[...], w_ref[...], (((1,), (1,)), ((), ())),
                        preferred_element_type=jnp.float32)
    y_ref[0] = y[:, :CH2]
    y_ref[1] = y[:, CH2:]


_mm = pl.pallas_call(
    _mm_body,
    grid=(GRID,),
    in_specs=[
        pl.BlockSpec((BLK, D), lambda i: (i, 0)),
        pl.BlockSpec((C, D), lambda i: (0, 0)),
    ],
    out_specs=pl.BlockSpec((NSC, BLK, CH2), lambda i: (0, i, 0)),
    out_shape=jax.ShapeDtypeStruct((NSC, NPAD, CH2), jnp.float32),
)


def kernel(x, edge_index, W, b):
    ei = edge_index.astype(jnp.int32)
    row = ei[0].reshape(NTILE, EPT)
    col = ei[1].reshape(NTILE, EPT)
    pad_r = jnp.zeros((NTILE, EPP), jnp.int32)
    pad_c = jnp.full((NTILE, EPP), DEAD, jnp.int32)
    row_s = jnp.concatenate([row, pad_r], axis=1).reshape(NTILE, NCH, CH)
    col_s = jnp.concatenate([col, pad_c], axis=1).reshape(NTILE, NCH, CH)

    ones = jnp.ones((CH,), jnp.float32)
    z1 = jnp.zeros((NPAD,), jnp.float32)

    y0 = _mm(x, W)
    out, _ = _sc_all(y0, row_s, col_s, ones, z1, b.reshape(NSC, CH2))
    return out


# speedup vs baseline: 1.2431x; 1.2431x over previous
import functools

import jax
import jax.numpy as jnp
from jax import lax
from jax.experimental import pallas as pl
from jax.experimental.pallas import tpu as pltpu
from jax.experimental.pallas import tpu_sc as plsc

N = 10000
E = 320000
D = 128
C = 64
CH2 = C // 2

NSC = 2
NTILE = 16
EPT = E // NTILE
CH = 128
NCH = EPT // CH + 1
EPP = NCH * CH - EPT
NPAD = 10240
RPT = NPAD // NTILE
DEAD = N + 200

_MESH = plsc.VectorSubcoreMesh(
    core_axis_name="c", subcore_axis_name="s",
    num_cores=NSC, num_subcores=NTILE)


def _rsqrt16(x):
    xi = plsc.bitcast(x, jnp.int32)
    yi = jnp.int32(0x5F3759DF) - lax.shift_right_logical(xi, 1)
    y = plsc.bitcast(yi, jnp.float32)
    for _ in range(3):
        y = y * (1.5 - 0.5 * x * y * y)
    return y


@functools.partial(
    pl.kernel,
    out_type=jax.ShapeDtypeStruct((N, C), jnp.float32),
    mesh=_MESH,
    compiler_params=pltpu.CompilerParams(use_tc_tiling_on_sc=False,
                                         needs_layout_passes=False),
    scratch_types=[
        pltpu.VMEM((NCH, CH), jnp.int32),
        pltpu.VMEM((NCH, CH), jnp.int32),
        pltpu.VMEM((CH,), jnp.float32),
        pltpu.VMEM((2, CH, CH2), jnp.float32),
        pltpu.VMEM((RPT, CH2), jnp.float32),
        pltpu.VMEM((RPT,), jnp.float32),
        pltpu.VMEM((RPT,), jnp.float32),
        pltpu.VMEM((RPT,), jnp.float32),
        pltpu.VMEM((CH2,), jnp.float32),
        pltpu.VMEM_SHARED((NPAD,), jnp.float32),
        pltpu.VMEM_SHARED((NPAD, CH2), jnp.float32),
        pltpu.VMEM_SHARED((NPAD, CH2), jnp.float32),
        pltpu.SemaphoreType.DMA,
        pltpu.SemaphoreType.DMA,
    ],
)
def _sc_all(y0_hbm, row_hbm, col_hbm, ones_hbm, z1_hbm, b_hbm, out_hbm,
            row_v, col_v, ones_v, pay, buf, degb, dinvb, d2b, bloc,
            deg_sp, u_sp, acc, sem0, sem1):
    c = lax.axis_index("c")
    s = lax.axis_index("s")
    base = s * RPT

    pltpu.sync_copy(row_hbm.at[s], row_v)
    pltpu.sync_copy(col_hbm.at[s], col_v)
    pltpu.sync_copy(ones_hbm, ones_v)
    pltpu.sync_copy(b_hbm.at[c], bloc)
    pltpu.sync_copy(z1_hbm.at[pl.ds(base, RPT)], deg_sp.at[pl.ds(base, RPT)])
    pltpu.sync_copy(y0_hbm.at[c, pl.ds(base, RPT)], buf)
    plsc.subcore_barrier()

    def dchunk(j, carry):
        pltpu.async_copy(ones_v, deg_sp.at[col_v.at[j]], sem0, add=True)
        return carry

    lax.fori_loop(0, NCH, dchunk, 0)

    def ddrain(j, carry):
        pltpu.make_async_copy(ones_v, deg_sp.at[col_v.at[0]], sem0).wait()
        return carry

    lax.fori_loop(0, NCH, ddrain, 0)
    plsc.subcore_barrier()

    pltpu.sync_copy(deg_sp.at[pl.ds(base, RPT)], degb)

    def newton(k, carry):
        x = degb[pl.ds(k * 16, 16)] + 1.0
        y = _rsqrt16(x)
        dinvb[pl.ds(k * 16, 16)] = y
        d2b[pl.ds(k * 16, 16)] = y * y
        return carry

    lax.fori_loop(0, RPT // 16, newton, 0)

    def uscale(g, carry):
        r0 = g * 16
        dv = dinvb[pl.ds(r0, 16)]
        for l in range(16):
            r = r0 + l
            buf[r, pl.ds(0, 16)] = buf[r, pl.ds(0, 16)] * dv[l]
            buf[r, pl.ds(16, 16)] = buf[r, pl.ds(16, 16)] * dv[l]
        return carry

    lax.fori_loop(0, RPT // 16, uscale, 0)
    pltpu.sync_copy(buf, u_sp.at[pl.ds(base, RPT)])
    pltpu.sync_copy(buf, acc.at[pl.ds(base, RPT)])
    plsc.subcore_barrier()

    def hop():
        pltpu.async_copy(u_sp.at[row_v.at[0]], pay.at[0], sem0)

        def pair(i, carry):
            j = i * 2
            pltpu.async_copy(u_sp.at[row_v.at[j + 1]], pay.at[1], sem1)
            pltpu.make_async_copy(u_sp.at[row_v.at[j]], pay.at[0],
                                  sem0).wait()
            pltpu.sync_copy(pay.at[0], acc.at[col_v.at[j]], add=True)
            pltpu.async_copy(u_sp.at[row_v.at[j + 2]], pay.at[0], sem0)
            pltpu.make_async_copy(u_sp.at[row_v.at[j + 1]], pay.at[1],
                                  sem1).wait()
            pltpu.sync_copy(pay.at[1], acc.at[col_v.at[j + 1]], add=True)
            return carry

        lax.fori_loop(0, (NCH - 1) // 2, pair, 0)
        pltpu.make_async_copy(u_sp.at[row_v.at[NCH - 1]], pay.at[0],
                              sem0).wait()
        pltpu.sync_copy(pay.at[0], acc.at[col_v.at[NCH - 1]], add=True)
        plsc.subcore_barrier()

    assert NCH % 2 == 1
    hop()

    pltpu.sync_copy(acc.at[pl.ds(base, RPT)], buf)

    def mscale(g, carry):
        r0 = g * 16
        dv = d2b[pl.ds(r0, 16)]
        for l in range(16):
            r = r0 + l
            buf[r, pl.ds(0, 16)] = buf[r, pl.ds(0, 16)] * dv[l]
            buf[r, pl.ds(16, 16)] = buf[r, pl.ds(16, 16)] * dv[l]
        return carry

    lax.fori_loop(0, RPT // 16, mscale, 0)
    pltpu.sync_copy(buf, u_sp.at[pl.ds(base, RPT)])
    pltpu.sync_copy(buf, acc.at[pl.ds(base, RPT)])
    plsc.subcore_barrier()

    hop()

    pltpu.sync_copy(acc.at[pl.ds(base, RPT)], buf)
    b0 = bloc[pl.ds(0, 16)]
    b1 = bloc[pl.ds(16, 16)]

    def oscale(g, carry):
        r0 = g * 16
        dv = dinvb[pl.ds(r0, 16)]
        for l in range(16):
            r = r0 + l
            buf[r, pl.ds(0, 16)] = buf[r, pl.ds(0, 16)] * dv[l] + b0
            buf[r, pl.ds(16, 16)] = buf[r, pl.ds(16, 16)] * dv[l] + b1
        return carry

    lax.fori_loop(0, RPT // 16, oscale, 0)
    last = N - (NTILE - 1) * RPT

    @pl.when(s < NTILE - 1)
    def _():
        pltpu.sync_copy(buf,
                        out_hbm.at[pl.ds(base, RPT), pl.ds(c * CH2, CH2)])

    @pl.when(s == NTILE - 1)
    def _():
        pltpu.sync_copy(buf.at[pl.ds(0, last)],
                        out_hbm.at[pl.ds(base, last), pl.ds(c * CH2, CH2)])


BLK = 80
GRID = N // BLK


def _mm_body(x_ref, w_ref, y_ref):
    y = lax.dot_general(x_ref[...], w_ref[...], (((1,), (1,)), ((), ())),
                        preferred_element_type=jnp.float32)
    y_ref[0] = y[:, :CH2]
    y_ref[1] = y[:, CH2:]


_mm = pl.pallas_call(
    _mm_body,
    grid=(GRID,),
    in_specs=[
        pl.BlockSpec((BLK, D), lambda i: (i, 0)),
        pl.BlockSpec((C, D), lambda i: (0, 0)),
    ],
    out_specs=pl.BlockSpec((NSC, BLK, CH2), lambda i: (0, i, 0)),
    out_shape=jax.ShapeDtypeStruct((NSC, NPAD, CH2), jnp.float32),
)


def kernel(x, edge_index, W, b):
    ei = edge_index.astype(jnp.int32)
    row = ei[0].reshape(NTILE, EPT)
    col = ei[1].reshape(NTILE, EPT)
    pad_r = jnp.zeros((NTILE, EPP), jnp.int32)
    pad_c = jnp.full((NTILE, EPP), DEAD, jnp.int32)
    row_s = jnp.concatenate([row, pad_r], axis=1).reshape(NTILE, NCH, CH)
    col_s = jnp.concatenate([col, pad_c], axis=1).reshape(NTILE, NCH, CH)

    ones = jnp.ones((CH,), jnp.float32)
    z1 = jnp.zeros((NPAD,), jnp.float32)

    y0 = _mm(x, W)
    return _sc_all(y0, row_s, col_s, ones, z1, b.reshape(NSC, CH2))
